# baseline (speedup 1.0000x reference)
import functools

import jax
import jax.numpy as jnp
from jax import lax
from jax.experimental import pallas as pl
from jax.experimental.pallas import tpu as pltpu
from jax.experimental.pallas import tpu_sc as plsc

B = 16384
DIM = 32
TW = 128
W = 8
NSLOT = 2 * W


def _detile(wid, tab_hbm, lin_hbm, staging, rsems, wsems, nt_full, vocab,
            tpw):
    base = wid * tpw
    limit = jnp.minimum(base + tpw, nt_full)
    nwaves = (tpw + W - 1) // W
    if nwaves % 2:
        nwaves += 1

    def slot(h, s):
        return staging.at[h * W + s]

    def read_desc(t, h, s, sem):
        return pltpu.make_async_copy(
            tab_hbm.at[:, pl.ds(t * TW, TW)], slot(h, s), sem)

    for s in range(W):
        t0 = base + s

        @pl.when(t0 < limit)
        def _():
            read_desc(t0, 0, s, rsems[0]).start()

    def half_body(wv, h):
        tw0 = base + wv * W
        for s in range(W):
            t = tw0 + s

            @pl.when(t < limit)
            def _():
                read_desc(t, h, s, rsems[h]).wait()

        for s in range(W):
            t2 = tw0 + W + s

            @pl.when(t2 < limit)
            def _():
                read_desc(t2, 1 - h, s, rsems[1 - h]).start()

        for s in range(W):
            t = tw0 + s

            @pl.when(t < limit)
            def _():
                for j in range(DIM):
                    pltpu.async_copy(
                        slot(h, s).at[j],
                        lin_hbm.at[pl.ds(j * vocab + t * TW, TW)],
                        wsems[h])

        for s in range(W):
            t = tw0 + s

            @pl.when(t < limit)
            def _():
                pltpu.make_async_copy(
                    tab_hbm.at[:, pl.ds(0, TW)], slot(h, s),
                    wsems[h]).wait()

    def body(w2, carry):
        half_body(w2 * 2, 0)
        half_body(w2 * 2 + 1, 1)
        return carry

    lax.fori_loop(0, nwaves // 2, body, 0)


def kernel(user_id, item_id, user_table, item_table, bias):
    info = plsc.get_sparse_core_info()
    nw = info.num_cores * info.num_subcores
    bw = B // nw
    nblk = bw // 16
    uv = user_table.shape[0]
    iv = item_table.shape[0]
    u_full = uv // TW
    i_full = iv // TW
    u_tpw = -(-u_full // nw)
    i_tpw = -(-i_full // nw)

    uid = jnp.reshape(user_id, (B,))
    iid = jnp.reshape(item_id, (B,))
    ut = jnp.transpose(user_table)
    it = jnp.transpose(item_table)
    utail = jnp.reshape(jnp.transpose(user_table[u_full * TW:, :]), (-1,))
    itail = jnp.reshape(jnp.transpose(item_table[i_full * TW:, :]), (-1,))
    u_rem = uv - u_full * TW
    i_rem = iv - i_full * TW
    bias16 = jnp.broadcast_to(jnp.reshape(bias, (1,)), (16,))

    mesh = plsc.VectorSubcoreMesh(core_axis_name="c", subcore_axis_name="s")

    @functools.partial(
        pl.kernel,
        out_type=(jax.ShapeDtypeStruct((DIM * uv,), jnp.float32),
                  jax.ShapeDtypeStruct((DIM * iv,), jnp.float32)),
        mesh=mesh,
        compiler_params=pltpu.CompilerParams(needs_layout_passes=False),
        scratch_types=[
            pltpu.VMEM((NSLOT, DIM, TW), jnp.float32),
            pltpu.VMEM((DIM * TW,), jnp.float32),
            pltpu.SemaphoreType.DMA,
            pltpu.SemaphoreType.DMA,
            pltpu.SemaphoreType.DMA,
            pltpu.SemaphoreType.DMA,
        ],
    )
    def k1(ut_hbm, it_hbm, utail_hbm, itail_hbm, ulin_hbm, ilin_hbm,
           staging, tailbuf, rs0, rs1, ws0, ws1):
        wid = lax.axis_index("s") * info.num_cores + lax.axis_index("c")
        _detile(wid, ut_hbm, ulin_hbm, staging, (rs0, rs1), (ws0, ws1),
                u_full, uv, u_tpw)
        _detile(wid, it_hbm, ilin_hbm, staging, (rs0, rs1), (ws0, ws1),
                i_full, iv, i_tpw)

        @pl.when(wid == nw - 1)
        def _():
            pltpu.sync_copy(utail_hbm, tailbuf.at[pl.ds(0, DIM * u_rem)])
            for j in range(DIM):
                pltpu.sync_copy(
                    tailbuf.at[pl.ds(j * u_rem, u_rem)],
                    ulin_hbm.at[pl.ds(j * uv + u_full * TW, u_rem)])
            pltpu.sync_copy(itail_hbm, tailbuf.at[pl.ds(0, DIM * i_rem)])
            for j in range(DIM):
                pltpu.sync_copy(
                    tailbuf.at[pl.ds(j * i_rem, i_rem)],
                    ilin_hbm.at[pl.ds(j * iv + i_full * TW, i_rem)])

    @functools.partial(
        pl.kernel,
        out_type=jax.ShapeDtypeStruct((2 * B,), jnp.float32),
        mesh=mesh,
        compiler_params=pltpu.CompilerParams(needs_layout_passes=False),
        scratch_types=[
            pltpu.VMEM((bw,), jnp.int32),
            pltpu.VMEM((bw,), jnp.int32),
            pltpu.VMEM((bw * DIM,), jnp.float32),
            pltpu.VMEM((bw * DIM,), jnp.float32),
            pltpu.VMEM((2 * bw,), jnp.float32),
            pltpu.VMEM((16,), jnp.float32),
            pltpu.SemaphoreType.DMA,
            pltpu.SemaphoreType.DMA,
        ],
    )
    def k2(uid_hbm, iid_hbm, ulin_hbm, ilin_hbm, bias_hbm, out_hbm,
           idx_u, idx_i, ubuf, ibuf, outb, bias_v, semu, semi):
        wid = lax.axis_index("s") * info.num_cores + lax.axis_index("c")
        base = wid * bw

        pltpu.sync_copy(uid_hbm.at[pl.ds(base, bw)], idx_u)
        pltpu.sync_copy(iid_hbm.at[pl.ds(base, bw)], idx_i)
        pltpu.sync_copy(bias_hbm, bias_v)

        def fire(k, carry):
            ivu = idx_u[pl.ds(k * 16, 16)]
            ivi = idx_i[pl.ds(k * 16, 16)]
            for j in range(DIM):
                s = pl.ds((k * DIM + j) * 16, 16)
                pltpu.async_copy(ulin_hbm.at[ivu + j * uv], ubuf.at[s], semu)
                pltpu.async_copy(ilin_hbm.at[ivi + j * iv], ibuf.at[s], semi)
            return carry

        lax.fori_loop(0, nblk, fire, 0)

        pltpu.make_async_copy(
            ulin_hbm.at[pl.ds(0, bw * DIM)], ubuf, semu).wait()
        pltpu.make_async_copy(
            ilin_hbm.at[pl.ds(0, bw * DIM)], ibuf, semi).wait()

        bias_vec = bias_v[...]
        lanes = lax.iota(jnp.int32, 16)
        zeros16 = jnp.zeros((16,), jnp.float32)

        def body(k, carry):
            acc = jnp.zeros((16,), jnp.float32)
            for j in range(DIM):
                s = pl.ds((k * DIM + j) * 16, 16)
                acc = acc + ubuf[s] * ibuf[s]
            pos = (k * 16 + lanes) * 2
            plsc.store_scatter(outb, [pos], zeros16)
            plsc.store_scatter(outb, [pos + 1], acc + bias_vec)
            return carry

        lax.fori_loop(0, nblk, body, 0)

        pltpu.sync_copy(outb, out_hbm.at[pl.ds(2 * base, 2 * bw)])

    ulin, ilin = k1(ut, it, utail, itail)
    logits = k2(uid, iid, ulin, ilin, bias16)
    return jnp.reshape(logits, (B, 2))


# speedup vs baseline: 1.0730x; 1.0730x over previous
import functools

import jax
import jax.numpy as jnp
from jax import lax
from jax.experimental import pallas as pl
from jax.experimental.pallas import tpu as pltpu
from jax.experimental.pallas import tpu_sc as plsc

B = 16384
DIM = 32
TW = 128
W = 8
NSLOT = 2 * W


def _detile(wid, tab_hbm, lin_hbm, staging, rsems, wsems, nt_full, vocab,
            tpw):
    base = wid * tpw
    limit = jnp.minimum(base + tpw, nt_full)
    nwaves = (tpw + W - 1) // W
    if nwaves % 2:
        nwaves += 1

    def slot(h, s):
        return staging.at[h * W + s]

    def read_desc(t, h, s, sem):
        return pltpu.make_async_copy(
            tab_hbm.at[:, pl.ds(t * TW, TW)], slot(h, s), sem)

    for s in range(W):
        t0 = base + s

        @pl.when(t0 < limit)
        def _():
            read_desc(t0, 0, s, rsems[0]).start()

    def half_body(wv, h):
        tw0 = base + wv * W
        for s in range(W):
            t = tw0 + s

            @pl.when(t < limit)
            def _():
                read_desc(t, h, s, rsems[h]).wait()

        for s in range(W):
            t2 = tw0 + W + s

            @pl.when(t2 < limit)
            def _():
                read_desc(t2, 1 - h, s, rsems[1 - h]).start()

        for s in range(W):
            t = tw0 + s

            @pl.when(t < limit)
            def _():
                for j in range(DIM):
                    pltpu.async_copy(
                        slot(h, s).at[j],
                        lin_hbm.at[pl.ds(j * vocab + t * TW, TW)],
                        wsems[h])

        for s in range(W):
            t = tw0 + s

            @pl.when(t < limit)
            def _():
                pltpu.make_async_copy(
                    tab_hbm.at[:, pl.ds(0, TW)], slot(h, s),
                    wsems[h]).wait()

    def body(w2, carry):
        half_body(w2 * 2, 0)
        half_body(w2 * 2 + 1, 1)
        return carry

    lax.fori_loop(0, nwaves // 2, body, 0)


def kernel(user_id, item_id, user_table, item_table, bias):
    info = plsc.get_sparse_core_info()
    nw = info.num_cores * info.num_subcores
    bw = B // nw
    nblk = bw // 16
    uv = user_table.shape[0]
    iv = item_table.shape[0]
    u_full = uv // TW
    i_full = iv // TW
    u_tpw = -(-u_full // nw)
    i_tpw = -(-i_full // nw)

    uid = jnp.reshape(user_id, (B,))
    iid = jnp.reshape(item_id, (B,))
    ut = jnp.transpose(user_table)
    it = jnp.transpose(item_table)
    utail = jnp.reshape(jnp.transpose(user_table[u_full * TW:, :]), (-1,))
    itail = jnp.reshape(jnp.transpose(item_table[i_full * TW:, :]), (-1,))
    u_rem = uv - u_full * TW
    i_rem = iv - i_full * TW
    bias16 = jnp.broadcast_to(jnp.reshape(bias, (1,)), (16,))

    mesh = plsc.VectorSubcoreMesh(core_axis_name="c", subcore_axis_name="s")

    @functools.partial(
        pl.kernel,
        out_type=(jax.ShapeDtypeStruct((DIM * uv,), jnp.float32),
                  jax.ShapeDtypeStruct((DIM * iv,), jnp.float32)),
        mesh=mesh,
        compiler_params=pltpu.CompilerParams(needs_layout_passes=False),
        scratch_types=[
            pltpu.VMEM((NSLOT, DIM, TW), jnp.float32),
            pltpu.VMEM((DIM * TW,), jnp.float32),
            pltpu.SemaphoreType.DMA,
            pltpu.SemaphoreType.DMA,
            pltpu.SemaphoreType.DMA,
            pltpu.SemaphoreType.DMA,
        ],
    )
    def k1(ut_hbm, it_hbm, utail_hbm, itail_hbm, ulin_hbm, ilin_hbm,
           staging, tailbuf, rs0, rs1, ws0, ws1):
        wid = lax.axis_index("s") * info.num_cores + lax.axis_index("c")
        _detile(wid, ut_hbm, ulin_hbm, staging, (rs0, rs1), (ws0, ws1),
                u_full, uv, u_tpw)
        _detile(wid, it_hbm, ilin_hbm, staging, (rs0, rs1), (ws0, ws1),
                i_full, iv, i_tpw)

        @pl.when(wid == nw - 1)
        def _():
            pltpu.sync_copy(utail_hbm, tailbuf.at[pl.ds(0, DIM * u_rem)])
            for j in range(DIM):
                pltpu.sync_copy(
                    tailbuf.at[pl.ds(j * u_rem, u_rem)],
                    ulin_hbm.at[pl.ds(j * uv + u_full * TW, u_rem)])
            pltpu.sync_copy(itail_hbm, tailbuf.at[pl.ds(0, DIM * i_rem)])
            for j in range(DIM):
                pltpu.sync_copy(
                    tailbuf.at[pl.ds(j * i_rem, i_rem)],
                    ilin_hbm.at[pl.ds(j * iv + i_full * TW, i_rem)])

    @functools.partial(
        pl.kernel,
        out_type=jax.ShapeDtypeStruct((B,), jnp.float32),
        mesh=mesh,
        compiler_params=pltpu.CompilerParams(needs_layout_passes=False),
        scratch_types=[
            pltpu.VMEM((bw,), jnp.int32),
            pltpu.VMEM((bw,), jnp.int32),
            pltpu.VMEM((bw * DIM,), jnp.float32),
            pltpu.VMEM((bw * DIM,), jnp.float32),
            pltpu.VMEM((bw,), jnp.float32),
            pltpu.VMEM((16,), jnp.float32),
            pltpu.SemaphoreType.DMA,
            pltpu.SemaphoreType.DMA,
        ],
    )
    def k2(uid_hbm, iid_hbm, ulin_hbm, ilin_hbm, bias_hbm, out_hbm,
           idx_u, idx_i, ubuf, ibuf, outb, bias_v, semu, semi):
        wid = lax.axis_index("s") * info.num_cores + lax.axis_index("c")
        base = wid * bw

        pltpu.sync_copy(uid_hbm.at[pl.ds(base, bw)], idx_u)
        pltpu.sync_copy(iid_hbm.at[pl.ds(base, bw)], idx_i)
        pltpu.sync_copy(bias_hbm, bias_v)

        def fire(k, carry):
            ivu = idx_u[pl.ds(k * 16, 16)]
            ivi = idx_i[pl.ds(k * 16, 16)]
            for j in range(DIM):
                s = pl.ds((k * DIM + j) * 16, 16)
                pltpu.async_copy(ulin_hbm.at[ivu + j * uv], ubuf.at[s], semu)
                pltpu.async_copy(ilin_hbm.at[ivi + j * iv], ibuf.at[s], semi)
            return carry

        lax.fori_loop(0, nblk, fire, 0)

        pltpu.make_async_copy(
            ulin_hbm.at[pl.ds(0, bw * DIM)], ubuf, semu).wait()
        pltpu.make_async_copy(
            ilin_hbm.at[pl.ds(0, bw * DIM)], ibuf, semi).wait()

        bias_vec = bias_v[...]

        def body(k, carry):
            acc = jnp.zeros((16,), jnp.float32)
            for j in range(DIM):
                s = pl.ds((k * DIM + j) * 16, 16)
                acc = acc + ubuf[s] * ibuf[s]
            outb[pl.ds(k * 16, 16)] = acc + bias_vec
            return carry

        lax.fori_loop(0, nblk, body, 0)

        pltpu.sync_copy(outb, out_hbm.at[pl.ds(base, bw)])

    ulin, ilin = k1(ut, it, utail, itail)
    layer = k2(uid, iid, ulin, ilin, bias16)
    return jnp.concatenate(
        [jnp.zeros((B, 1), jnp.float32), jnp.reshape(layer, (B, 1))], axis=1)
